# baseline (speedup 1.0000x reference)
import functools

import jax
import jax.numpy as jnp
from jax import lax
from jax.experimental import pallas as pl
from jax.experimental.pallas import tpu as pltpu
from jax.experimental.pallas import tpu_sc as plsc

_B, _P, _D = 64, 1024, 768
_NC, _NS, _L = 2, 16, 16
_NW = _NC * _NS
_PW = _P // _NW
_HR = _PW // 2
_NCHUNK = _D // _L
_NBUF = 8
_LA = 4
_NU = 2 * _B


def _compute(x_v, pos_v, h):
    def row_body(r, carry):
        pr = h * _HR + r
        for c in range(_NCHUNK):
            sl = pl.ds(c * _L, _L)
            plsc.addupdate(x_v.at[r, sl], pos_v[pr, sl])
        return carry

    lax.fori_loop(0, _HR, row_body, 0)


def _sc_kernel_body(x_hbm, pos_hbm, out_hbm, pos_v, bufs, in_sems, out_sems):
    wid = lax.axis_index("s") * _NC + lax.axis_index("c")
    ps = wid * _PW
    psl = pl.ds(ps, _PW)

    def rows(h):
        return pl.ds(ps + h * _HR, _HR)

    def step(u, k, h, first, last):
        k2 = (k + _LA) % _NBUF
        b = u // 2
        pltpu.make_async_copy(x_hbm.at[b, rows(h)], bufs[k], in_sems[k]).wait()
        _compute(bufs[k], pos_v, h)
        pltpu.async_copy(bufs[k], out_hbm.at[b, rows(h)], out_sems[k])
        if not last:
            if not first:
                bp = (u - _LA) // 2
                pltpu.make_async_copy(
                    bufs[k2], out_hbm.at[bp, rows(h)], out_sems[k2]).wait()
            bn = (u + _LA) // 2
            pltpu.async_copy(x_hbm.at[bn, rows(h)], bufs[k2], in_sems[k2])

    pltpu.sync_copy(pos_hbm.at[psl], pos_v)
    for u in range(_LA):
        pltpu.async_copy(x_hbm.at[u // 2, rows(u % 2)], bufs[u], in_sems[u])

    for u in range(_LA):
        step(u, u, u % 2, first=True, last=False)

    def group_body(i, carry):
        u0 = _NBUF * i + _LA
        for j in range(_NBUF):
            step(u0 + j, (_LA + j) % _NBUF, j % 2, first=False, last=False)
        return carry

    lax.fori_loop(0, (_NU - 2 * _LA) // _NBUF, group_body, 0)

    for j in range(_LA):
        u = _NU - _LA + j
        step(u, u % _NBUF, u % 2, first=False, last=True)

    for j in range(_NBUF):
        u = _NU - _NBUF + j
        pltpu.make_async_copy(
            bufs[u % _NBUF], out_hbm.at[u // 2, rows(u % 2)],
            out_sems[u % _NBUF]).wait()


@functools.partial(
    pl.kernel,
    out_type=jax.ShapeDtypeStruct((_B, _P, _D), jnp.float32),
    mesh=plsc.VectorSubcoreMesh(
        core_axis_name="c", subcore_axis_name="s",
        num_cores=_NC, num_subcores=_NS,
    ),
    scratch_types=(
        [pltpu.VMEM((_PW, _D), jnp.float32)]
        + [pltpu.VMEM((_HR, _D), jnp.float32)] * _NBUF
        + [pltpu.SemaphoreType.DMA] * (2 * _NBUF)
    ),
)
def _sc_kernel(x_hbm, pos_hbm, out_hbm, pos_v, *rest):
    bufs = rest[:_NBUF]
    in_sems = rest[_NBUF:2 * _NBUF]
    out_sems = rest[2 * _NBUF:]
    _sc_kernel_body(x_hbm, pos_hbm, out_hbm, pos_v, bufs, in_sems, out_sems)


def kernel(encoded_patches, pos_table):
    return _sc_kernel(encoded_patches, pos_table)


# speedup vs baseline: 1.2545x; 1.2545x over previous
import jax
import jax.numpy as jnp
from jax.experimental import pallas as pl

_B, _P, _D = 64, 1024, 768
_RB = 512


def _tc_body(x_ref, p_ref, o_ref):
    o_ref[...] = x_ref[...] + p_ref[...]


def kernel(encoded_patches, pos_table):
    x2 = encoded_patches.reshape(_B * _P, _D)
    out = pl.pallas_call(
        _tc_body,
        grid=(_P // _RB, _B),
        in_specs=[
            pl.BlockSpec((_RB, _D), lambda i, j: (j * (_P // _RB) + i, 0)),
            pl.BlockSpec((_RB, _D), lambda i, j: (i, 0)),
        ],
        out_specs=pl.BlockSpec((_RB, _D), lambda i, j: (j * (_P // _RB) + i, 0)),
        out_shape=jax.ShapeDtypeStruct((_B * _P, _D), jnp.float32),
    )(x2, pos_table)
    return out.reshape(_B, _P, _D)


# speedup vs baseline: 1.5636x; 1.2464x over previous
import jax
import jax.numpy as jnp
from jax.experimental import pallas as pl

_B, _P, _D = 64, 1024, 768
_BPB = 2


def _tc_body(x_ref, p_ref, o_ref):
    for b in range(_BPB):
        sl = slice(b * _P, (b + 1) * _P)
        o_ref[sl, :] = x_ref[sl, :] + p_ref[...]


def kernel(encoded_patches, pos_table):
    x2 = encoded_patches.reshape(_B * _P, _D)
    out = pl.pallas_call(
        _tc_body,
        grid=(_B // _BPB,),
        in_specs=[
            pl.BlockSpec((_BPB * _P, _D), lambda i: (i, 0)),
            pl.BlockSpec((_P, _D), lambda i: (0, 0)),
        ],
        out_specs=pl.BlockSpec((_BPB * _P, _D), lambda i: (i, 0)),
        out_shape=jax.ShapeDtypeStruct((_B * _P, _D), jnp.float32),
    )(x2, pos_table)
    return out.reshape(_B, _P, _D)
